# baseline (speedup 1.0000x reference)
import functools

import jax
import jax.numpy as jnp
from jax import lax
from jax.experimental import pallas as pl
from jax.experimental.pallas import tpu as pltpu
from jax.experimental.pallas import tpu_sc as plsc

V = 8192
NTOK = 8192
NW = 32
ROWS_PER_W = NTOK // NW
K = 8

_mesh = plsc.VectorSubcoreMesh(core_axis_name="c", subcore_axis_name="s")


@functools.partial(
    pl.kernel,
    mesh=_mesh,
    out_type=jax.ShapeDtypeStruct((NTOK, V), jnp.float32),
    scratch_types=[
        pltpu.VMEM((ROWS_PER_W,), jnp.int32),
    ]
    + [pltpu.SemaphoreType.DMA for _ in range(K)],
)
def _gather_kernel(ids_hbm, table_hbm, out_hbm, idx_v, *sems):
    wid = lax.axis_index("s") * 2 + lax.axis_index("c")
    base = wid * ROWS_PER_W
    pltpu.sync_copy(ids_hbm.at[wid], idx_v)

    def start_row(row, i, k):
        pltpu.make_async_copy(
            table_hbm.at[pl.ds(row, 1)],
            out_hbm.at[pl.ds(base + i, 1)],
            sems[k],
        ).start()

    def drain(k):
        pltpu.make_async_copy(
            table_hbm.at[pl.ds(0, 1)],
            out_hbm.at[pl.ds(base, 1)],
            sems[k],
        ).wait()

    v0 = idx_v[pl.ds(0, 16)]
    for lane in range(16):
        if lane >= K:
            drain(lane % K)
        start_row(v0[lane], lane, lane % K)

    def outer(g, carry):
        vg = idx_v[pl.ds(g * 16, 16)]
        for lane in range(16):
            drain(lane % K)
            start_row(vg[lane], g * 16 + lane, lane % K)
        return carry

    lax.fori_loop(1, ROWS_PER_W // 16, outer, 0)

    for k in range(K):
        drain(k)


def kernel(ids, table):
    B, T = ids.shape
    ids2 = ids.reshape(NW, ROWS_PER_W).astype(jnp.int32)
    out = _gather_kernel(ids2, table)
    return out.reshape(B, T, V)
